# baseline (speedup 1.0000x reference)
import functools

import jax
import jax.numpy as jnp
from jax import lax
from jax.experimental import pallas as pl
from jax.experimental.pallas import tpu as pltpu
from jax.experimental.pallas import tpu_sc as plsc

N = 10000
E = 320000
F_IN = 128
H = 128
C = 64

BLK = 256
NP = 10240
NBLK = NP // BLK
DW = 16

NC = 2
NS = 16
NW = NC * NS

CH = 128
KCH = 16
EPW = 10240
EPT = EPW * NW
ETW = EPT // NS
NG0 = EPW // (KCH * CH)
NG1 = ETW // (KCH * CH)
RPT = NP // NS


def _zero_fill(buf, nrows, ncols):
    zero16 = jnp.zeros((16,), jnp.float32)

    def zrow(i, _):
        for c in range(ncols // 16):
            buf[i, pl.ds(c * 16, 16)] = zero16
        return _

    lax.fori_loop(0, nrows, zrow, None)


def _load_idx_batch(arr, base, dst2d, sem):
    hs = []
    for k in range(KCH):
        hs.append(pltpu.async_copy(
            arr.at[pl.ds(base + k * CH, CH)], dst2d.at[k], sem))
    return hs


GSZ = KCH * CH


def _pipelined_gather_scatter(table, idx_arr, ibase, dstp, dbase,
                              src_v, dst_v, rows, acc,
                              sem_i, sem_g, sem_s, ngroups):
    def group(g, _):
        goff = g * GSZ
        ih = _load_idx_batch(idx_arr, ibase + goff, src_v, sem_i)
        dh = _load_idx_batch(dstp, dbase + goff, dst_v, sem_i)
        for h in ih + dh:
            h.wait()
        hg = [None, None]
        hs = [None, None]
        hg[0] = pltpu.async_copy(table.at[src_v.at[0]], rows[0], sem_g[0])
        for k in range(KCH):
            b = k & 1
            nb = 1 - b
            if k + 1 < KCH:
                if hs[nb] is not None:
                    hs[nb].wait()
                hg[nb] = pltpu.async_copy(
                    table.at[src_v.at[k + 1]], rows[nb], sem_g[nb])
            hg[b].wait()
            hs[b] = pltpu.async_copy(rows[b], acc.at[dst_v.at[k]],
                                     sem_s[b], add=True)
        hs[0].wait()
        hs[1].wait()
        return _

    lax.fori_loop(0, ngroups, group, None)


def _make_agg0():
    mesh = plsc.VectorSubcoreMesh(core_axis_name="c", subcore_axis_name="s")

    @functools.partial(
        pl.kernel,
        out_type=(
            jax.ShapeDtypeStruct((2 * NP, H), jnp.float32),
            jax.ShapeDtypeStruct((2 * NP, H), jnp.float32),
        ),
        mesh=mesh,
        scratch_types=[
            pltpu.VMEM((KCH, CH), jnp.int32),
            pltpu.VMEM((KCH, CH), jnp.int32),
            pltpu.VMEM((CH, H), jnp.float32),
            pltpu.VMEM((CH, H), jnp.float32),
            pltpu.VMEM_SHARED((NP, H), jnp.float32),
            pltpu.SemaphoreType.DMA,
            pltpu.SemaphoreType.DMA,
            pltpu.SemaphoreType.DMA,
            pltpu.SemaphoreType.DMA,
            pltpu.SemaphoreType.DMA,
        ],
    )
    def agg0(h0, srcp, dstp, out, dout, src_v, dst_v,
             rows0, rows1, acc, sem_i, sem_g0, sem_g1, sem_s0, sem_s1):
        cid = lax.axis_index("c")
        sid = lax.axis_index("s")
        wid = cid * NS + sid
        ebase = wid * EPW
        obase = cid * NP + sid * RPT

        _zero_fill(rows0, CH, H)
        for t in range(RPT // CH):
            pltpu.sync_copy(rows0, acc.at[pl.ds(sid * RPT + t * CH, CH)])
        one16 = jnp.ones((16,), jnp.float32)

        def orow(i, _):
            for c in range(H // 16):
                rows1[i, pl.ds(c * 16, 16)] = one16
            return _

        lax.fori_loop(0, CH, orow, None)
        plsc.subcore_barrier()

        def dgroup(g, _):
            dh = _load_idx_batch(dstp, ebase + g * GSZ, dst_v, sem_i)
            for h in dh:
                h.wait()
            hs = []
            for k in range(KCH):
                hs.append(pltpu.async_copy(
                    rows1, acc.at[dst_v.at[k]], sem_s0, add=True))
            for h in hs:
                h.wait()
            return _

        lax.fori_loop(0, NG0, dgroup, None)
        plsc.subcore_barrier()
        pltpu.sync_copy(acc.at[pl.ds(sid * RPT, RPT)],
                        dout.at[pl.ds(obase, RPT)])

        for t in range(RPT // CH):
            pltpu.sync_copy(rows0, acc.at[pl.ds(sid * RPT + t * CH, CH)])
        plsc.subcore_barrier()

        _pipelined_gather_scatter(
            h0, srcp, ebase, dstp, ebase,
            src_v, dst_v, (rows0, rows1), acc,
            sem_i, (sem_g0, sem_g1), (sem_s0, sem_s1), NG0)

        plsc.subcore_barrier()
        pltpu.sync_copy(acc.at[pl.ds(sid * RPT, RPT)],
                        out.at[pl.ds(obase, RPT)])

    return agg0


def _make_agg1():
    mesh = plsc.VectorSubcoreMesh(core_axis_name="c", subcore_axis_name="s")

    @functools.partial(
        pl.kernel,
        out_type=jax.ShapeDtypeStruct((2 * NP, H), jnp.float32),
        mesh=mesh,
        scratch_types=[
            pltpu.VMEM((KCH, CH), jnp.int32),
            pltpu.VMEM((KCH, CH), jnp.int32),
            pltpu.VMEM((CH, H), jnp.float32),
            pltpu.VMEM((CH, H), jnp.float32),
            pltpu.VMEM_SHARED((NP, H), jnp.float32),
            pltpu.SemaphoreType.DMA,
            pltpu.SemaphoreType.DMA,
            pltpu.SemaphoreType.DMA,
            pltpu.SemaphoreType.DMA,
            pltpu.SemaphoreType.DMA,
        ],
    )
    def agg1(h1cat, src2, dstp, out, src_v, dst_v,
             rows0, rows1, acc, sem_i, sem_g0, sem_g1, sem_s0, sem_s1):
        cid = lax.axis_index("c")
        sid = lax.axis_index("s")
        ebase = sid * ETW

        _zero_fill(rows0, CH, H)
        for t in range(RPT // CH):
            pltpu.sync_copy(rows0, acc.at[pl.ds(sid * RPT + t * CH, CH)])
        plsc.subcore_barrier()

        _pipelined_gather_scatter(
            h1cat, src2, cid * EPT + ebase, dstp, ebase,
            src_v, dst_v, (rows0, rows1), acc,
            sem_i, (sem_g0, sem_g1), (sem_s0, sem_s1), NG1)

        plsc.subcore_barrier()
        pltpu.sync_copy(
            acc.at[pl.ds(sid * RPT, RPT)],
            out.at[pl.ds(cid * NP + sid * RPT, RPT)],
        )

    return agg1


def _row_mask(pid):
    rows = pid * BLK + lax.broadcasted_iota(jnp.int32, (BLK, 1), 0)
    return rows < N


def _k1_body(x_ref, w_ref, b_ref, out_ref):
    h = jnp.maximum(
        jnp.dot(x_ref[...], w_ref[...], preferred_element_type=jnp.float32)
        + b_ref[...],
        0.0,
    )
    out_ref[...] = jnp.where(_row_mask(pl.program_id(0)), h, 0.0)


def _k2_body(p0_ref, p1_ref, d0_ref, d1_ref, h0_ref, ws_ref, wn_ref, b_ref,
             p_ref, stats_ref, inv_ref):
    s = p0_ref[...] + p1_ref[...]
    deg = d0_ref[:, 0:1] + d1_ref[:, 0:1]
    inv = 1.0 / jnp.maximum(deg, 1.0)
    hn = s * inv
    p = (
        jnp.dot(h0_ref[...], ws_ref[...], preferred_element_type=jnp.float32)
        + jnp.dot(hn, wn_ref[...], preferred_element_type=jnp.float32)
        + b_ref[...]
    )
    p = jnp.where(_row_mask(pl.program_id(0)), p, 0.0)
    p_ref[...] = p
    ssum = jnp.sum(p, axis=0).reshape(1, 1, 2 * H)
    ssq = jnp.sum(p * p, axis=0).reshape(1, 1, 2 * H)
    stats_ref[...] = jnp.concatenate([ssum, ssq], axis=1)
    inv_ref[...] = jnp.broadcast_to(inv, (BLK, H))


def _k3_body(p_ref, stats_ref, g_ref, be_ref, out_ref):
    st = stats_ref[...]
    mu = jnp.sum(st[:, 0, :], axis=0) / N
    var = jnp.sum(st[:, 1, :], axis=0) / N - mu * mu
    rstd = lax.rsqrt(var + 1e-5)
    h = jnp.maximum(g_ref[...] * (p_ref[...] - mu[None, :]) * rstd[None, :]
                    + be_ref[...], 0.0)
    h = jnp.where(_row_mask(pl.program_id(0)), h, 0.0)
    out_ref[0] = h[:, :H]
    out_ref[1] = h[:, H:]


def _k4_body(h1a_ref, h1b_ref, s0_ref, s1_ref, inv_ref,
             wsa_ref, wsb_ref, wna_ref, wnb_ref, b_ref,
             q_ref, stats_ref):
    inv = inv_ref[...]
    q = (
        jnp.dot(h1a_ref[0], wsa_ref[...], preferred_element_type=jnp.float32)
        + jnp.dot(h1b_ref[0], wsb_ref[...], preferred_element_type=jnp.float32)
        + jnp.dot(s0_ref[...] * inv, wna_ref[...], preferred_element_type=jnp.float32)
        + jnp.dot(s1_ref[...] * inv, wnb_ref[...], preferred_element_type=jnp.float32)
        + b_ref[...]
    )
    q = jnp.where(_row_mask(pl.program_id(0)), q, 0.0)
    q_ref[...] = q
    ssum = jnp.sum(q, axis=0).reshape(1, 1, H)
    ssq = jnp.sum(q * q, axis=0).reshape(1, 1, H)
    stats_ref[...] = jnp.concatenate([ssum, ssq], axis=1)


def _k5_body(q_ref, stats_ref, g_ref, be_ref, h0_ref,
             wrt_ref, wrb_ref, br_ref, wc1_ref, bc1_ref, wc2_ref, bc2_ref,
             out_ref):
    st = stats_ref[...]
    mu = jnp.sum(st[:, 0, :], axis=0) / N
    var = jnp.sum(st[:, 1, :], axis=0) / N - mu * mu
    rstd = lax.rsqrt(var + 1e-5)
    h2 = jnp.maximum(g_ref[...] * (q_ref[...] - mu[None, :]) * rstd[None, :]
                     + be_ref[...], 0.0)
    hf = jnp.maximum(
        jnp.dot(h0_ref[...], wrt_ref[...], preferred_element_type=jnp.float32)
        + jnp.dot(h2, wrb_ref[...], preferred_element_type=jnp.float32)
        + br_ref[...],
        0.0,
    )
    t = jnp.maximum(
        jnp.dot(hf, wc1_ref[...], preferred_element_type=jnp.float32)
        + bc1_ref[...],
        0.0,
    )
    out_ref[...] = (
        jnp.dot(t, wc2_ref[...], preferred_element_type=jnp.float32)
        + bc2_ref[...]
    )


def _full(shape):
    return pl.BlockSpec(shape, lambda i: tuple(0 for _ in shape))


def _rows(width):
    return pl.BlockSpec((BLK, width), lambda i: (i, 0))


_agg_cache = {}


def _agg0_call(h0, srcp, dstp):
    if "a0" not in _agg_cache:
        _agg_cache["a0"] = _make_agg0()
    return _agg_cache["a0"](h0, srcp, dstp)


def _agg1_call(h1cat, src2, dstp):
    if "a1" not in _agg_cache:
        _agg_cache["a1"] = _make_agg1()
    return _agg_cache["a1"](h1cat, src2, dstp)


def kernel(features, edge_index, W_in, b_in, Ws0, Wn0, b0, g0, be0,
           Ws1, Wn1, b1, g1, be1, W_rel, b_rel, Wc1, bc1, Wc2, bc2):
    f32 = jnp.float32

    src = edge_index[0]
    dst = edge_index[1]
    npad = EPT - E
    srcp = jnp.concatenate([src, jnp.full((npad,), N, jnp.int32)])
    dstp = jnp.concatenate([dst, jnp.full((npad,), N, jnp.int32),
                            jnp.zeros((GSZ,), jnp.int32)])
    src2 = jnp.concatenate([srcp, srcp + NP, jnp.zeros((GSZ,), jnp.int32)])
    srcp = jnp.concatenate([srcp, jnp.zeros((GSZ,), jnp.int32)])

    wrt = W_rel[:H]
    wrb = W_rel[H:]
    ws1a, ws1b = Ws1[:H], Ws1[H:]
    wn1a, wn1b = Wn1[:H], Wn1[H:]

    b_in2 = b_in.reshape(1, H)
    b02 = b0.reshape(1, 2 * H)
    g02 = g0.reshape(1, 2 * H)
    be02 = be0.reshape(1, 2 * H)
    b12 = b1.reshape(1, H)
    g12 = g1.reshape(1, H)
    be12 = be1.reshape(1, H)
    br2 = b_rel.reshape(1, H)
    bc12 = bc1.reshape(1, H // 2)
    bc22 = bc2.reshape(1, C)

    h0 = pl.pallas_call(
        _k1_body,
        grid=(NBLK,),
        in_specs=[_rows(F_IN), _full((F_IN, H)), _full((1, H))],
        out_specs=_rows(H),
        out_shape=jax.ShapeDtypeStruct((NP, H), f32),
    )(features, W_in, b_in2)

    part, degp = _agg0_call(h0, srcp, dstp)

    p, stats0, inv_deg = pl.pallas_call(
        _k2_body,
        grid=(NBLK,),
        in_specs=[
            pl.BlockSpec((BLK, H), lambda i: (i, 0)),
            pl.BlockSpec((BLK, H), lambda i: (i + NBLK, 0)),
            pl.BlockSpec((BLK, H), lambda i: (i, 0)),
            pl.BlockSpec((BLK, H), lambda i: (i + NBLK, 0)),
            _rows(H),
            _full((H, 2 * H)), _full((H, 2 * H)), _full((1, 2 * H)),
        ],
        out_specs=[
            _rows(2 * H),
            pl.BlockSpec((1, 2, 2 * H), lambda i: (i, 0, 0)),
            _rows(H),
        ],
        out_shape=[
            jax.ShapeDtypeStruct((NP, 2 * H), f32),
            jax.ShapeDtypeStruct((NBLK, 2, 2 * H), f32),
            jax.ShapeDtypeStruct((NP, H), f32),
        ],
    )(part, part, degp, degp, h0, Ws0, Wn0, b02)

    h1 = pl.pallas_call(
        _k3_body,
        grid=(NBLK,),
        in_specs=[
            _rows(2 * H),
            _full((NBLK, 2, 2 * H)),
            _full((1, 2 * H)), _full((1, 2 * H)),
        ],
        out_specs=pl.BlockSpec((2, BLK, H), lambda i: (0, i, 0)),
        out_shape=jax.ShapeDtypeStruct((2, NP, H), f32),
    )(p, stats0, g02, be02)

    h1cat = h1.reshape(2 * NP, H)

    sums1 = _agg1_call(h1cat, src2, dstp)

    q, stats1 = pl.pallas_call(
        _k4_body,
        grid=(NBLK,),
        in_specs=[
            pl.BlockSpec((1, BLK, H), lambda i: (0, i, 0)),
            pl.BlockSpec((1, BLK, H), lambda i: (1, i, 0)),
            pl.BlockSpec((BLK, H), lambda i: (i, 0)),
            pl.BlockSpec((BLK, H), lambda i: (i + NBLK, 0)),
            _rows(H),
            _full((H, H)), _full((H, H)), _full((H, H)), _full((H, H)),
            _full((1, H)),
        ],
        out_specs=[
            _rows(H),
            pl.BlockSpec((1, 2, H), lambda i: (i, 0, 0)),
        ],
        out_shape=[
            jax.ShapeDtypeStruct((NP, H), f32),
            jax.ShapeDtypeStruct((NBLK, 2, H), f32),
        ],
    )(h1, h1, sums1, sums1, inv_deg, ws1a, ws1b, wn1a, wn1b, b12)

    out = pl.pallas_call(
        _k5_body,
        grid=(NBLK,),
        in_specs=[
            _rows(H),
            _full((NBLK, 2, H)),
            _full((1, H)), _full((1, H)),
            _rows(H),
            _full((H, H)), _full((H, H)), _full((1, H)),
            _full((H, H // 2)), _full((1, H // 2)),
            _full((H // 2, C)), _full((1, C)),
        ],
        out_specs=_rows(C),
        out_shape=jax.ShapeDtypeStruct((NP, C), f32),
    )(q, stats1, g12, be12, h0, wrt, wrb, br2, Wc1, bc12, Wc2, bc22)

    return out[:N]


# speedup vs baseline: 1.2284x; 1.2284x over previous
import functools

import jax
import jax.numpy as jnp
from jax import lax
from jax.experimental import pallas as pl
from jax.experimental.pallas import tpu as pltpu
from jax.experimental.pallas import tpu_sc as plsc

N = 10000
E = 320000
F_IN = 128
H = 128
C = 64

BLK = 256
NP = 10240
NBLK = NP // BLK
DW = 16

NC = 2
NS = 16
NW = NC * NS

CH = 128
KCH = 16
EPW = 10240
EPT = EPW * NW
ETW = EPT // NS
NG0 = EPW // (KCH * CH)
NG1 = ETW // (KCH * CH)
RPT = NP // NS


def _zero_fill(buf, nrows, ncols):
    zero16 = jnp.zeros((16,), jnp.float32)

    def zrow(i, _):
        for c in range(ncols // 16):
            buf[i, pl.ds(c * 16, 16)] = zero16
        return _

    lax.fori_loop(0, nrows, zrow, None)


def _load_idx_batch(arr, base, dst2d, sem):
    hs = []
    for k in range(KCH):
        hs.append(pltpu.async_copy(
            arr.at[pl.ds(base + k * CH, CH)], dst2d.at[k], sem))
    return hs


GSZ = KCH * CH


def _pipelined_gather_scatter(table, idx_arr, ibase, dstp, dbase,
                              src_v, dst_v, rows, acc,
                              sem_i, sem_g, sem_s, ngroups):
    def group(g, _):
        goff = g * GSZ
        ih = _load_idx_batch(idx_arr, ibase + goff, src_v, sem_i)
        dh = _load_idx_batch(dstp, dbase + goff, dst_v, sem_i)
        for h in ih + dh:
            h.wait()
        hg = [None, None]
        hs = [None, None]
        hg[0] = pltpu.async_copy(table.at[src_v.at[0]], rows[0], sem_g[0])
        for k in range(KCH):
            b = k & 1
            nb = 1 - b
            if k + 1 < KCH:
                if hs[nb] is not None:
                    hs[nb].wait()
                hg[nb] = pltpu.async_copy(
                    table.at[src_v.at[k + 1]], rows[nb], sem_g[nb])
            hg[b].wait()
            hs[b] = pltpu.async_copy(rows[b], acc.at[dst_v.at[k]],
                                     sem_s[b], add=True)
        hs[0].wait()
        hs[1].wait()
        return _

    lax.fori_loop(0, ngroups, group, None)


def _make_agg0():
    mesh = plsc.VectorSubcoreMesh(core_axis_name="c", subcore_axis_name="s")

    @functools.partial(
        pl.kernel,
        out_type=(
            jax.ShapeDtypeStruct((2 * NP, H), jnp.float32),
            jax.ShapeDtypeStruct((2 * NP, H), jnp.float32),
        ),
        mesh=mesh,
        scratch_types=[
            pltpu.VMEM((KCH, CH), jnp.int32),
            pltpu.VMEM((KCH, CH), jnp.int32),
            pltpu.VMEM((CH, H), jnp.float32),
            pltpu.VMEM((CH, H), jnp.float32),
            pltpu.VMEM_SHARED((NP, H), jnp.float32),
            pltpu.SemaphoreType.DMA,
            pltpu.SemaphoreType.DMA,
            pltpu.SemaphoreType.DMA,
            pltpu.SemaphoreType.DMA,
            pltpu.SemaphoreType.DMA,
        ],
    )
    def agg0(h0, srcp, dstp, out, dout, src_v, dst_v,
             rows0, rows1, acc, sem_i, sem_g0, sem_g1, sem_s0, sem_s1):
        cid = lax.axis_index("c")
        sid = lax.axis_index("s")
        wid = cid * NS + sid
        ebase = wid * EPW
        obase = cid * NP + sid * RPT

        _zero_fill(rows0, CH, H)
        for t in range(RPT // CH):
            pltpu.sync_copy(rows0, acc.at[pl.ds(sid * RPT + t * CH, CH)])
        one16 = jnp.ones((16,), jnp.float32)

        def orow(i, _):
            for c in range(H // 16):
                rows1[i, pl.ds(c * 16, 16)] = one16
            return _

        lax.fori_loop(0, CH, orow, None)
        plsc.subcore_barrier()

        def dgroup(g, _):
            dh = _load_idx_batch(dstp, ebase + g * GSZ, dst_v, sem_i)
            for h in dh:
                h.wait()
            hs = []
            for k in range(KCH):
                hs.append(pltpu.async_copy(
                    rows1, acc.at[dst_v.at[k]], sem_s0, add=True))
            for h in hs:
                h.wait()
            return _

        lax.fori_loop(0, NG0, dgroup, None)
        plsc.subcore_barrier()
        pltpu.sync_copy(acc.at[pl.ds(sid * RPT, RPT)],
                        dout.at[pl.ds(obase, RPT)])

        for t in range(RPT // CH):
            pltpu.sync_copy(rows0, acc.at[pl.ds(sid * RPT + t * CH, CH)])
        plsc.subcore_barrier()

        _pipelined_gather_scatter(
            h0, srcp, ebase, dstp, ebase,
            src_v, dst_v, (rows0, rows1), acc,
            sem_i, (sem_g0, sem_g1), (sem_s0, sem_s1), NG0)

        plsc.subcore_barrier()
        pltpu.sync_copy(acc.at[pl.ds(sid * RPT, RPT)],
                        out.at[pl.ds(obase, RPT)])

    return agg0


def _make_agg1():
    mesh = plsc.VectorSubcoreMesh(core_axis_name="c", subcore_axis_name="s")

    @functools.partial(
        pl.kernel,
        out_type=jax.ShapeDtypeStruct((2 * NP, H), jnp.float32),
        mesh=mesh,
        scratch_types=[
            pltpu.VMEM((KCH, CH), jnp.int32),
            pltpu.VMEM((KCH, CH), jnp.int32),
            pltpu.VMEM((CH, H), jnp.float32),
            pltpu.VMEM((CH, H), jnp.float32),
            pltpu.VMEM_SHARED((NP, H), jnp.float32),
            pltpu.SemaphoreType.DMA,
            pltpu.SemaphoreType.DMA,
            pltpu.SemaphoreType.DMA,
            pltpu.SemaphoreType.DMA,
            pltpu.SemaphoreType.DMA,
        ],
    )
    def agg1(h1cat, src2, dstp, out, src_v, dst_v,
             rows0, rows1, acc, sem_i, sem_g0, sem_g1, sem_s0, sem_s1):
        cid = lax.axis_index("c")
        sid = lax.axis_index("s")
        ebase = sid * ETW

        _zero_fill(rows0, CH, H)
        for t in range(RPT // CH):
            pltpu.sync_copy(rows0, acc.at[pl.ds(sid * RPT + t * CH, CH)])
        plsc.subcore_barrier()

        _pipelined_gather_scatter(
            h1cat, src2, cid * EPT + ebase, dstp, ebase,
            src_v, dst_v, (rows0, rows1), acc,
            sem_i, (sem_g0, sem_g1), (sem_s0, sem_s1), NG1)

        plsc.subcore_barrier()
        pltpu.sync_copy(
            acc.at[pl.ds(sid * RPT, RPT)],
            out.at[pl.ds(cid * NP + sid * RPT, RPT)],
        )

    return agg1


def _row_mask(pid):
    rows = pid * BLK + lax.broadcasted_iota(jnp.int32, (BLK, 1), 0)
    return rows < N


def _k1_body(x_ref, w_ref, b_ref, out_ref):
    h = jnp.maximum(
        jnp.dot(x_ref[...], w_ref[...], preferred_element_type=jnp.float32)
        + b_ref[...],
        0.0,
    )
    out_ref[...] = jnp.where(_row_mask(pl.program_id(0)), h, 0.0)


def _k2_body(p0_ref, p1_ref, d0_ref, d1_ref, h0_ref, ws_ref, wn_ref, b_ref,
             p_ref, stats_ref, inv_ref):
    s = p0_ref[...] + p1_ref[...]
    deg = d0_ref[:, 0:1] + d1_ref[:, 0:1]
    inv = 1.0 / jnp.maximum(deg, 1.0)
    hn = s * inv
    p = (
        jnp.dot(h0_ref[...], ws_ref[...], preferred_element_type=jnp.float32)
        + jnp.dot(hn, wn_ref[...], preferred_element_type=jnp.float32)
        + b_ref[...]
    )
    p = jnp.where(_row_mask(pl.program_id(0)), p, 0.0)
    p_ref[...] = p
    ssum = jnp.sum(p, axis=0).reshape(1, 1, 2 * H)
    ssq = jnp.sum(p * p, axis=0).reshape(1, 1, 2 * H)
    stats_ref[...] = jnp.concatenate([ssum, ssq], axis=1)
    inv_ref[...] = jnp.broadcast_to(inv, (BLK, H))


def _k3_body(p_ref, stats_ref, g_ref, be_ref, out_ref):
    st = stats_ref[...]
    mu = jnp.sum(st[:, 0, :], axis=0) / N
    var = jnp.sum(st[:, 1, :], axis=0) / N - mu * mu
    rstd = lax.rsqrt(var + 1e-5)
    h = jnp.maximum(g_ref[...] * (p_ref[...] - mu[None, :]) * rstd[None, :]
                    + be_ref[...], 0.0)
    h = jnp.where(_row_mask(pl.program_id(0)), h, 0.0)
    out_ref[0] = h[:, :H]
    out_ref[1] = h[:, H:]


def _k4_body(h1a_ref, h1b_ref, s0_ref, s1_ref, inv_ref,
             wsa_ref, wsb_ref, wna_ref, wnb_ref, b_ref,
             q_ref, stats_ref):
    inv = inv_ref[...]
    q = (
        jnp.dot(h1a_ref[0], wsa_ref[...], preferred_element_type=jnp.float32)
        + jnp.dot(h1b_ref[0], wsb_ref[...], preferred_element_type=jnp.float32)
        + jnp.dot(s0_ref[...] * inv, wna_ref[...], preferred_element_type=jnp.float32)
        + jnp.dot(s1_ref[...] * inv, wnb_ref[...], preferred_element_type=jnp.float32)
        + b_ref[...]
    )
    q = jnp.where(_row_mask(pl.program_id(0)), q, 0.0)
    q_ref[...] = q
    ssum = jnp.sum(q, axis=0).reshape(1, 1, H)
    ssq = jnp.sum(q * q, axis=0).reshape(1, 1, H)
    stats_ref[...] = jnp.concatenate([ssum, ssq], axis=1)


def _k5_body(q_ref, stats_ref, g_ref, be_ref, h0_ref,
             wrt_ref, wrb_ref, br_ref, wc1_ref, bc1_ref, wc2_ref, bc2_ref,
             out_ref):
    st = stats_ref[...]
    mu = jnp.sum(st[:, 0, :], axis=0) / N
    var = jnp.sum(st[:, 1, :], axis=0) / N - mu * mu
    rstd = lax.rsqrt(var + 1e-5)
    h2 = jnp.maximum(g_ref[...] * (q_ref[...] - mu[None, :]) * rstd[None, :]
                     + be_ref[...], 0.0)
    hf = jnp.maximum(
        jnp.dot(h0_ref[...], wrt_ref[...], preferred_element_type=jnp.float32)
        + jnp.dot(h2, wrb_ref[...], preferred_element_type=jnp.float32)
        + br_ref[...],
        0.0,
    )
    t = jnp.maximum(
        jnp.dot(hf, wc1_ref[...], preferred_element_type=jnp.float32)
        + bc1_ref[...],
        0.0,
    )
    out_ref[...] = (
        jnp.dot(t, wc2_ref[...], preferred_element_type=jnp.float32)
        + bc2_ref[...]
    )


def _full(shape):
    return pl.BlockSpec(shape, lambda i: tuple(0 for _ in shape))


def _rows(width):
    return pl.BlockSpec((BLK, width), lambda i: (i, 0))


_agg_cache = {}


def _agg0_call(h0, srcp, dstp):
    if "a0" not in _agg_cache:
        _agg_cache["a0"] = _make_agg0()
    return _agg_cache["a0"](h0, srcp, dstp)


def _agg1_call(h1cat, src2, dstp):
    if "a1" not in _agg_cache:
        _agg_cache["a1"] = _make_agg1()
    return _agg_cache["a1"](h1cat, src2, dstp)


def kernel(features, edge_index, W_in, b_in, Ws0, Wn0, b0, g0, be0,
           Ws1, Wn1, b1, g1, be1, W_rel, b_rel, Wc1, bc1, Wc2, bc2):
    f32 = jnp.float32

    src = edge_index[0]
    dst = edge_index[1]
    npad = EPT - E
    srcp = jnp.concatenate([src, jnp.full((npad,), N, jnp.int32)])
    dstp = jnp.concatenate([dst, jnp.full((npad,), N, jnp.int32)])
    src2 = jnp.concatenate([srcp, srcp + NP])

    wrt = W_rel[:H]
    wrb = W_rel[H:]
    ws1a, ws1b = Ws1[:H], Ws1[H:]
    wn1a, wn1b = Wn1[:H], Wn1[H:]

    b_in2 = b_in.reshape(1, H)
    b02 = b0.reshape(1, 2 * H)
    g02 = g0.reshape(1, 2 * H)
    be02 = be0.reshape(1, 2 * H)
    b12 = b1.reshape(1, H)
    g12 = g1.reshape(1, H)
    be12 = be1.reshape(1, H)
    br2 = b_rel.reshape(1, H)
    bc12 = bc1.reshape(1, H // 2)
    bc22 = bc2.reshape(1, C)

    h0 = pl.pallas_call(
        _k1_body,
        grid=(NBLK,),
        in_specs=[_rows(F_IN), _full((F_IN, H)), _full((1, H))],
        out_specs=_rows(H),
        out_shape=jax.ShapeDtypeStruct((NP, H), f32),
    )(features, W_in, b_in2)

    part, degp = _agg0_call(h0, srcp, dstp)

    p, stats0, inv_deg = pl.pallas_call(
        _k2_body,
        grid=(NBLK,),
        in_specs=[
            pl.BlockSpec((BLK, H), lambda i: (i, 0)),
            pl.BlockSpec((BLK, H), lambda i: (i + NBLK, 0)),
            pl.BlockSpec((BLK, H), lambda i: (i, 0)),
            pl.BlockSpec((BLK, H), lambda i: (i + NBLK, 0)),
            _rows(H),
            _full((H, 2 * H)), _full((H, 2 * H)), _full((1, 2 * H)),
        ],
        out_specs=[
            _rows(2 * H),
            pl.BlockSpec((1, 2, 2 * H), lambda i: (i, 0, 0)),
            _rows(H),
        ],
        out_shape=[
            jax.ShapeDtypeStruct((NP, 2 * H), f32),
            jax.ShapeDtypeStruct((NBLK, 2, 2 * H), f32),
            jax.ShapeDtypeStruct((NP, H), f32),
        ],
    )(part, part, degp, degp, h0, Ws0, Wn0, b02)

    h1 = pl.pallas_call(
        _k3_body,
        grid=(NBLK,),
        in_specs=[
            _rows(2 * H),
            _full((NBLK, 2, 2 * H)),
            _full((1, 2 * H)), _full((1, 2 * H)),
        ],
        out_specs=pl.BlockSpec((2, BLK, H), lambda i: (0, i, 0)),
        out_shape=jax.ShapeDtypeStruct((2, NP, H), f32),
    )(p, stats0, g02, be02)

    h1cat = h1.reshape(2 * NP, H)

    sums1 = _agg1_call(h1cat, src2, dstp)

    q, stats1 = pl.pallas_call(
        _k4_body,
        grid=(NBLK,),
        in_specs=[
            pl.BlockSpec((1, BLK, H), lambda i: (0, i, 0)),
            pl.BlockSpec((1, BLK, H), lambda i: (1, i, 0)),
            pl.BlockSpec((BLK, H), lambda i: (i, 0)),
            pl.BlockSpec((BLK, H), lambda i: (i + NBLK, 0)),
            _rows(H),
            _full((H, H)), _full((H, H)), _full((H, H)), _full((H, H)),
            _full((1, H)),
        ],
        out_specs=[
            _rows(H),
            pl.BlockSpec((1, 2, H), lambda i: (i, 0, 0)),
        ],
        out_shape=[
            jax.ShapeDtypeStruct((NP, H), f32),
            jax.ShapeDtypeStruct((NBLK, 2, H), f32),
        ],
    )(h1, h1, sums1, sums1, inv_deg, ws1a, ws1b, wn1a, wn1b, b12)

    out = pl.pallas_call(
        _k5_body,
        grid=(NBLK,),
        in_specs=[
            _rows(H),
            _full((NBLK, 2, H)),
            _full((1, H)), _full((1, H)),
            _rows(H),
            _full((H, H)), _full((H, H)), _full((1, H)),
            _full((H, H // 2)), _full((1, H // 2)),
            _full((H // 2, C)), _full((1, C)),
        ],
        out_specs=_rows(C),
        out_shape=jax.ShapeDtypeStruct((NP, C), f32),
    )(q, stats1, g12, be12, h0, wrt, wrb, br2, Wc1, bc12, Wc2, bc22)

    return out[:N]
